# Initial kernel scaffold; baseline (speedup 1.0000x reference)
import jax
import jax.numpy as jnp
from jax.experimental import pallas as pl


def kernel(in_feat, segment_ids, weight, bias):
    raise NotImplementedError("write your pallas kernel here")


# speedup vs baseline: 16.7648x; 16.7648x over previous
import functools

import jax
import jax.numpy as jnp
from jax.experimental import pallas as pl

NSEG = 16
DFEAT = 128
BLOCK = 6400


def _stats_kernel(seg_ref, x_ref, sums_ref, sumsq_ref, counts_ref):
    i = pl.program_id(0)
    x = x_ref[...]
    seg = seg_ref[0, 0, :]
    onehot = (seg[:, None] == jax.lax.broadcasted_iota(
        jnp.int32, (x.shape[0], NSEG), 1)).astype(jnp.float32)
    dim = (((0,), (0,)), ((), ()))
    psums = jax.lax.dot_general(onehot, x, dim,
                                preferred_element_type=jnp.float32)
    psumsq = jax.lax.dot_general(onehot, x * x, dim,
                                 preferred_element_type=jnp.float32)
    pcounts = jnp.broadcast_to(jnp.sum(onehot, axis=0)[:, None],
                               (NSEG, DFEAT))

    @pl.when(i == 0)
    def _init():
        sums_ref[...] = psums
        sumsq_ref[...] = psumsq
        counts_ref[...] = pcounts

    @pl.when(i > 0)
    def _acc():
        sums_ref[...] += psums
        sumsq_ref[...] += psumsq
        counts_ref[...] += pcounts


def _apply_kernel(seg_ref, x_ref, sums_ref, sumsq_ref, counts_ref,
                  w_ref, b_ref, out_ref):
    x = x_ref[...]
    seg = seg_ref[0, 0, :]
    n = jnp.maximum(counts_ref[...], 1.0)
    mean = sums_ref[...] / n
    var = sumsq_ref[...] / n - mean * mean
    inv = jax.lax.rsqrt(jnp.maximum(var, 0.0) + 1e-8)
    scale = inv * w_ref[...]
    shift = b_ref[...] - mean * scale
    onehot = (seg[:, None] == jax.lax.broadcasted_iota(
        jnp.int32, (x.shape[0], NSEG), 1)).astype(jnp.float32)
    rowscale = jnp.dot(onehot, scale, preferred_element_type=jnp.float32)
    rowshift = jnp.dot(onehot, shift, preferred_element_type=jnp.float32)
    out_ref[...] = x * rowscale + rowshift


@functools.partial(jax.jit, static_argnames=("interpret",))
def _run(in_feat, segment_ids, weight, bias, interpret=False):
    n_rows, d = in_feat.shape
    nblk = n_rows // BLOCK
    seg3d = segment_ids.astype(jnp.int32).reshape(nblk, 1, BLOCK)

    stats_shape = jax.ShapeDtypeStruct((NSEG, DFEAT), jnp.float32)
    sums, sumsq, counts = pl.pallas_call(
        _stats_kernel,
        grid=(nblk,),
        in_specs=[
            pl.BlockSpec((1, 1, BLOCK), lambda i: (i, 0, 0)),
            pl.BlockSpec((BLOCK, d), lambda i: (i, 0)),
        ],
        out_specs=[
            pl.BlockSpec((NSEG, DFEAT), lambda i: (0, 0)),
            pl.BlockSpec((NSEG, DFEAT), lambda i: (0, 0)),
            pl.BlockSpec((NSEG, DFEAT), lambda i: (0, 0)),
        ],
        out_shape=[stats_shape, stats_shape, stats_shape],
        interpret=interpret,
    )(seg3d, in_feat)

    out = pl.pallas_call(
        _apply_kernel,
        grid=(nblk,),
        in_specs=[
            pl.BlockSpec((1, 1, BLOCK), lambda i: (i, 0, 0)),
            pl.BlockSpec((BLOCK, d), lambda i: (i, 0)),
            pl.BlockSpec((NSEG, DFEAT), lambda i: (0, 0)),
            pl.BlockSpec((NSEG, DFEAT), lambda i: (0, 0)),
            pl.BlockSpec((NSEG, DFEAT), lambda i: (0, 0)),
            pl.BlockSpec((1, DFEAT), lambda i: (0, 0)),
            pl.BlockSpec((1, DFEAT), lambda i: (0, 0)),
        ],
        out_specs=pl.BlockSpec((BLOCK, d), lambda i: (i, 0)),
        out_shape=jax.ShapeDtypeStruct((n_rows, d), jnp.float32),
        interpret=interpret,
    )(seg3d, in_feat, sums, sumsq, counts, weight, bias)
    return out


def kernel(in_feat, segment_ids, weight, bias):
    return _run(in_feat, segment_ids, weight, bias)
